# baseline (speedup 1.0000x reference)
import functools

import jax
import jax.numpy as jnp
from jax import lax
from jax.experimental import pallas as pl
from jax.experimental.pallas import tpu as pltpu
from jax.experimental.pallas import tpu_sc as plsc

B = 16384
GMF_DIM = 16
MLP_DIM = 64

NC = 2
NS = 16
NW = NC * NS
B_PER_W = B // NW
CHUNK = 128
NCHUNK = B_PER_W // CHUNK


def _sc_gather(user, item, ug, ig, um, im):
    mesh = plsc.VectorSubcoreMesh(core_axis_name="c", subcore_axis_name="s")

    @functools.partial(
        pl.kernel,
        mesh=mesh,
        compiler_params=pltpu.CompilerParams(use_tc_tiling_on_sc=False),
        out_type=[
            jax.ShapeDtypeStruct((B, GMF_DIM), jnp.float32),
            jax.ShapeDtypeStruct((B, GMF_DIM), jnp.float32),
            jax.ShapeDtypeStruct((B, MLP_DIM), jnp.float32),
            jax.ShapeDtypeStruct((B, MLP_DIM), jnp.float32),
        ],
        scratch_types=[
            pltpu.VMEM((NCHUNK, CHUNK), jnp.int32),
            pltpu.VMEM((NCHUNK, CHUNK), jnp.int32),
            pltpu.VMEM((B_PER_W, GMF_DIM), jnp.float32),
            pltpu.VMEM((B_PER_W, GMF_DIM), jnp.float32),
            pltpu.VMEM((B_PER_W, MLP_DIM), jnp.float32),
            pltpu.VMEM((B_PER_W, MLP_DIM), jnp.float32),
            pltpu.SemaphoreType.DMA,
        ],
    )
    def k(user_hbm, item_hbm, ug_hbm, ig_hbm, um_hbm, im_hbm,
          out_ug, out_ig, out_um, out_im,
          idx_u, idx_i, bug, big, bum, bim, sem):
        wid = lax.axis_index("s") * NC + lax.axis_index("c")
        base = wid * B_PER_W
        pltpu.sync_copy(user_hbm.at[wid], idx_u)
        pltpu.sync_copy(item_hbm.at[wid], idx_i)
        copies = []
        for j in range(NCHUNK):
            dst = pl.ds(j * CHUNK, CHUNK)
            copies.append(pltpu.async_copy(ug_hbm.at[idx_u.at[j]], bug.at[dst], sem))
            copies.append(pltpu.async_copy(ig_hbm.at[idx_i.at[j]], big.at[dst], sem))
            copies.append(pltpu.async_copy(um_hbm.at[idx_u.at[j]], bum.at[dst], sem))
            copies.append(pltpu.async_copy(im_hbm.at[idx_i.at[j]], bim.at[dst], sem))
        for c in copies:
            c.wait()
        row = pl.ds(base, B_PER_W)
        pltpu.sync_copy(bug, out_ug.at[row])
        pltpu.sync_copy(big, out_ig.at[row])
        pltpu.sync_copy(bum, out_um.at[row])
        pltpu.sync_copy(bim, out_im.at[row])

    user3 = user.reshape(NW, NCHUNK, CHUNK)
    item3 = item.reshape(NW, NCHUNK, CHUNK)
    return k(user3, item3, ug, ig, um, im)


def _mlp_body(eug, eig, eum, eim, w1u, w1i, b1, w2t, b2, w3t, b3,
              wpg, wph, bp, out_ref):
    gmf = eug[...] * eig[...]
    h = jnp.dot(eum[...], w1u[...], preferred_element_type=jnp.float32)
    h = h + jnp.dot(eim[...], w1i[...], preferred_element_type=jnp.float32)
    h = jnp.maximum(h + b1[...], 0.0)
    h = jnp.maximum(
        jnp.dot(h, w2t[...], preferred_element_type=jnp.float32) + b2[...], 0.0)
    h = jnp.maximum(
        jnp.dot(h, w3t[...], preferred_element_type=jnp.float32) + b3[...], 0.0)
    r = jnp.sum(gmf * wpg[...], axis=1) + jnp.sum(h * wph[...], axis=1)
    out_ref[...] = r + bp[0]


def kernel(user, item, embed_user_GMF, embed_item_GMF, embed_user_MLP,
           embed_item_MLP, W1, b1, W2, b2, W3, b3, Wp, bp):
    user = user.astype(jnp.int32)
    item = item.astype(jnp.int32)
    eug, eig, eum, eim = _sc_gather(
        user, item, embed_user_GMF, embed_item_GMF,
        embed_user_MLP, embed_item_MLP)

    blk = 2048
    grid = B // blk
    row_spec = lambda d: pl.BlockSpec((blk, d), lambda i: (i, 0))
    full = lambda shape: pl.BlockSpec(shape, lambda i: (0,) * len(shape))

    out = pl.pallas_call(
        _mlp_body,
        grid=(grid,),
        in_specs=[
            row_spec(GMF_DIM), row_spec(GMF_DIM),
            row_spec(MLP_DIM), row_spec(MLP_DIM),
            full((MLP_DIM, MLP_DIM)), full((MLP_DIM, MLP_DIM)), full((1, MLP_DIM)),
            full((MLP_DIM, 32)), full((1, 32)),
            full((32, GMF_DIM)), full((1, GMF_DIM)),
            full((1, GMF_DIM)), full((1, GMF_DIM)), full((1,)),
        ],
        out_specs=pl.BlockSpec((blk,), lambda i: (i,)),
        out_shape=jax.ShapeDtypeStruct((B,), jnp.float32),
    )(
        eug, eig, eum, eim,
        W1[:, :MLP_DIM].T, W1[:, MLP_DIM:].T, b1.reshape(1, -1),
        W2.T, b2.reshape(1, -1),
        W3.T, b3.reshape(1, -1),
        Wp[:, :GMF_DIM], Wp[:, GMF_DIM:], bp,
    )
    return out


# speedup vs baseline: 2.5580x; 2.5580x over previous
import functools

import jax
import jax.numpy as jnp
from jax import lax
from jax.experimental import pallas as pl
from jax.experimental.pallas import tpu as pltpu
from jax.experimental.pallas import tpu_sc as plsc

B = 16384
GMF_DIM = 16
MLP_DIM = 64
LANES = 128

NC = 2
NS = 16
NW = NC * NS
B_PER_W = B // NW
RING = 4
GROUPS = B_PER_W // RING


def _sc_gather(user, item, ugT, igT, umT, imT):
    mesh = plsc.VectorSubcoreMesh(core_axis_name="c", subcore_axis_name="s")

    @functools.partial(
        pl.kernel,
        mesh=mesh,
        compiler_params=pltpu.CompilerParams(needs_layout_passes=False),
        out_type=[
            jax.ShapeDtypeStruct((GMF_DIM, B), jnp.float32),
            jax.ShapeDtypeStruct((GMF_DIM, B), jnp.float32),
            jax.ShapeDtypeStruct((MLP_DIM, B), jnp.float32),
            jax.ShapeDtypeStruct((MLP_DIM, B), jnp.float32),
        ],
        scratch_types=[
            pltpu.VMEM((B_PER_W,), jnp.int32),
            pltpu.VMEM((B_PER_W,), jnp.int32),
            pltpu.VMEM((RING, GMF_DIM, LANES), jnp.float32),
            pltpu.VMEM((RING, MLP_DIM, LANES), jnp.float32),
            pltpu.VMEM((GMF_DIM, B_PER_W), jnp.float32),
            pltpu.VMEM((MLP_DIM, B_PER_W), jnp.float32),
            pltpu.SemaphoreType.DMA,
            pltpu.SemaphoreType.DMA,
            pltpu.SemaphoreType.DMA,
            pltpu.SemaphoreType.DMA,
        ],
    )
    def k(user_hbm, item_hbm, ug_hbm, ig_hbm, um_hbm, im_hbm,
          out_ug, out_ig, out_um, out_im,
          idx_u, idx_i, gring, mring, gstage, mstage, *sems):
        wid = lax.axis_index("s") * NC + lax.axis_index("c")
        base = wid * B_PER_W
        col = pl.ds(base, B_PER_W)
        pltpu.sync_copy(user_hbm.at[col], idx_u)
        pltpu.sync_copy(item_hbm.at[col], idx_i)
        rows16 = lax.iota(jnp.int32, 16)

        def scalar_at(idx, i):
            grp = pl.multiple_of((i >> 4) << 4, 16)
            w = idx[pl.ds(grp, 16)]
            return jnp.sum(jnp.where(rows16 == (i & 15), w, 0))

        def run_side(idx, gtbl, mtbl, gout, mout):
            def fire(i, s):
                v = scalar_at(idx, i)
                t = pl.multiple_of((v >> 7) << 7, LANES)
                src = pl.ds(t, LANES)
                pltpu.async_copy(gtbl.at[:, src], gring.at[s], sems[s])
                pltpu.async_copy(mtbl.at[:, src], mring.at[s], sems[s])

            def drain_extract(i, s):
                pltpu.make_async_copy(
                    gtbl.at[:, pl.ds(0, LANES)], gring.at[s], sems[s]).wait()
                pltpu.make_async_copy(
                    mtbl.at[:, pl.ds(0, LANES)], mring.at[s], sems[s]).wait()
                lane = jnp.full((16,), scalar_at(idx, i) & 127, jnp.int32)
                pos = jnp.full((16,), i, jnp.int32)
                val = plsc.load_gather(gring.at[s], [rows16, lane])
                plsc.store_scatter(gstage, [rows16, pos], val)
                for kk in range(MLP_DIM // 16):
                    r16 = rows16 + (16 * kk)
                    val = plsc.load_gather(mring.at[s], [r16, lane])
                    plsc.store_scatter(mstage, [r16, pos], val)

            for s in range(RING):
                fire(s, s)

            def body(g, _):
                for s in range(RING):
                    drain_extract((g - 1) * RING + s, s)

                @pl.when(g < GROUPS)
                def _fire():
                    for s in range(RING):
                        fire(g * RING + s, s)
                return _
            lax.fori_loop(1, GROUPS + 1, body, None)
            pltpu.sync_copy(gstage, gout.at[:, col])
            pltpu.sync_copy(mstage, mout.at[:, col])

        run_side(idx_u, ug_hbm, um_hbm, out_ug, out_um)
        run_side(idx_i, ig_hbm, im_hbm, out_ig, out_im)

    return k(user, item, ugT, igT, umT, imT)


def _mlp_body(ug, ig, um, im, w1u, w1i, b1, w2, b2, w3, b3,
              wpg, wph, bp, out_ref):
    h = jnp.dot(w1u[...], um[...], preferred_element_type=jnp.float32)
    h = h + jnp.dot(w1i[...], im[...], preferred_element_type=jnp.float32)
    h = jnp.maximum(h + b1[...], 0.0)
    h = jnp.maximum(
        jnp.dot(w2[...], h, preferred_element_type=jnp.float32) + b2[...], 0.0)
    h = jnp.maximum(
        jnp.dot(w3[...], h, preferred_element_type=jnp.float32) + b3[...], 0.0)
    g = ug[...] * ig[...]
    r = jnp.sum(g * wpg[...], axis=0) + jnp.sum(h * wph[...], axis=0)
    out_ref[...] = r + bp[0]


def kernel(user, item, embed_user_GMF, embed_item_GMF, embed_user_MLP,
           embed_item_MLP, W1, b1, W2, b2, W3, b3, Wp, bp):
    user = user.astype(jnp.int32)
    item = item.astype(jnp.int32)
    ug, ig, um, im = _sc_gather(
        user, item, embed_user_GMF.T, embed_item_GMF.T,
        embed_user_MLP.T, embed_item_MLP.T)

    blk = 2048
    grid = B // blk
    col_spec = lambda d: pl.BlockSpec((d, blk), lambda i: (0, i))
    full = lambda shape: pl.BlockSpec(shape, lambda i: (0,) * len(shape))

    out = pl.pallas_call(
        _mlp_body,
        grid=(grid,),
        in_specs=[
            col_spec(GMF_DIM), col_spec(GMF_DIM),
            col_spec(MLP_DIM), col_spec(MLP_DIM),
            full((MLP_DIM, MLP_DIM)), full((MLP_DIM, MLP_DIM)),
            full((MLP_DIM, 1)),
            full((32, MLP_DIM)), full((32, 1)),
            full((GMF_DIM, 32)), full((GMF_DIM, 1)),
            full((GMF_DIM, 1)), full((GMF_DIM, 1)), full((1,)),
        ],
        out_specs=pl.BlockSpec((blk,), lambda i: (i,)),
        out_shape=jax.ShapeDtypeStruct((B,), jnp.float32),
    )(
        ug, ig, um, im,
        W1[:, :MLP_DIM], W1[:, MLP_DIM:], b1.reshape(-1, 1),
        W2, b2.reshape(-1, 1),
        W3, b3.reshape(-1, 1),
        Wp[:, :GMF_DIM].reshape(-1, 1), Wp[:, GMF_DIM:].reshape(-1, 1), bp,
    )
    return out


# speedup vs baseline: 2.8742x; 1.1236x over previous
import functools

import jax
import jax.numpy as jnp
from jax import lax
from jax.experimental import pallas as pl
from jax.experimental.pallas import tpu as pltpu
from jax.experimental.pallas import tpu_sc as plsc

B = 16384
GMF_DIM = 16
MLP_DIM = 64
LANES = 128

NC = 2
NS = 16
NW = NC * NS
B_PER_W = B // NW
RING = 8
GROUPS = B_PER_W // RING
TILE_BYTES = (GMF_DIM + MLP_DIM) * LANES * 4


def _sc_gather(user, item, ugT, igT, umT, imT):
    mesh = plsc.VectorSubcoreMesh(core_axis_name="c", subcore_axis_name="s")

    @functools.partial(
        pl.kernel,
        mesh=mesh,
        compiler_params=pltpu.CompilerParams(needs_layout_passes=False),
        out_type=[
            jax.ShapeDtypeStruct((GMF_DIM, B), jnp.float32),
            jax.ShapeDtypeStruct((GMF_DIM, B), jnp.float32),
            jax.ShapeDtypeStruct((MLP_DIM, B), jnp.float32),
            jax.ShapeDtypeStruct((MLP_DIM, B), jnp.float32),
        ],
        scratch_types=[
            pltpu.VMEM((B_PER_W,), jnp.int32),
            pltpu.VMEM((B_PER_W,), jnp.int32),
            pltpu.VMEM((RING, GMF_DIM, LANES), jnp.float32),
            pltpu.VMEM((RING, MLP_DIM, LANES), jnp.float32),
            pltpu.VMEM((GMF_DIM, B_PER_W), jnp.float32),
            pltpu.VMEM((MLP_DIM, B_PER_W), jnp.float32),
        ] + [pltpu.SemaphoreType.DMA] * RING,
    )
    def k(user_hbm, item_hbm, ug_hbm, ig_hbm, um_hbm, im_hbm,
          out_ug, out_ig, out_um, out_im,
          idx_u, idx_i, gring, mring, gstage, mstage, *sems):
        wid = lax.axis_index("s") * NC + lax.axis_index("c")
        base = wid * B_PER_W
        col = pl.ds(base, B_PER_W)
        pltpu.sync_copy(user_hbm.at[col], idx_u)
        pltpu.sync_copy(item_hbm.at[col], idx_i)
        rows16 = lax.iota(jnp.int32, 16)

        def scalar_at(idx, i):
            grp = pl.multiple_of((i >> 4) << 4, 16)
            w = idx[pl.ds(grp, 16)]
            return jnp.sum(jnp.where(rows16 == (i & 15), w, 0))

        def run_side(idx, gtbl, mtbl, gout, mout):
            def fire(i, s):
                v = scalar_at(idx, i)
                t = pl.multiple_of((v >> 7) << 7, LANES)
                src = pl.ds(t, LANES)
                pltpu.async_copy(gtbl.at[:, src], gring.at[s], sems[s])
                pltpu.async_copy(mtbl.at[:, src], mring.at[s], sems[s])

            def drain_extract(i, s):
                pltpu.make_async_copy(
                    gtbl.at[:, pl.ds(0, LANES)], gring.at[s], sems[s]).wait()
                pltpu.make_async_copy(
                    mtbl.at[:, pl.ds(0, LANES)], mring.at[s], sems[s]).wait()
                lane = jnp.full((16,), scalar_at(idx, i) & 127, jnp.int32)
                pos = jnp.full((16,), i, jnp.int32)
                val = plsc.load_gather(gring.at[s], [rows16, lane])
                plsc.store_scatter(gstage, [rows16, pos], val)
                for kk in range(MLP_DIM // 16):
                    r16 = rows16 + (16 * kk)
                    val = plsc.load_gather(mring.at[s], [r16, lane])
                    plsc.store_scatter(mstage, [r16, pos], val)

            for s in range(RING):
                fire(s, s)

            def body(g, _):
                for s in range(RING):
                    drain_extract((g - 1) * RING + s, s)

                @pl.when(g < GROUPS)
                def _fire():
                    for s in range(RING):
                        fire(g * RING + s, s)
                return _
            lax.fori_loop(1, GROUPS + 1, body, None)
            pltpu.sync_copy(gstage, gout.at[:, col])
            pltpu.sync_copy(mstage, mout.at[:, col])

        run_side(idx_u, ug_hbm, um_hbm, out_ug, out_um)
        run_side(idx_i, ig_hbm, im_hbm, out_ig, out_im)

    return k(user, item, ugT, igT, umT, imT)


def _mlp_body(ug, ig, um, im, w1u, w1i, b1, w2, b2, w3, b3,
              wpg, wph, bp, out_ref):
    h = jnp.dot(w1u[...], um[...], preferred_element_type=jnp.float32)
    h = h + jnp.dot(w1i[...], im[...], preferred_element_type=jnp.float32)
    h = jnp.maximum(h + b1[...], 0.0)
    h = jnp.maximum(
        jnp.dot(w2[...], h, preferred_element_type=jnp.float32) + b2[...], 0.0)
    h = jnp.maximum(
        jnp.dot(w3[...], h, preferred_element_type=jnp.float32) + b3[...], 0.0)
    g = ug[...] * ig[...]
    r = jnp.sum(g * wpg[...], axis=0) + jnp.sum(h * wph[...], axis=0)
    out_ref[...] = r + bp[0]


def kernel(user, item, embed_user_GMF, embed_item_GMF, embed_user_MLP,
           embed_item_MLP, W1, b1, W2, b2, W3, b3, Wp, bp):
    user = user.astype(jnp.int32)
    item = item.astype(jnp.int32)
    ug, ig, um, im = _sc_gather(
        user, item, embed_user_GMF.T, embed_item_GMF.T,
        embed_user_MLP.T, embed_item_MLP.T)

    blk = 2048
    grid = B // blk
    col_spec = lambda d: pl.BlockSpec((d, blk), lambda i: (0, i))
    full = lambda shape: pl.BlockSpec(shape, lambda i: (0,) * len(shape))

    out = pl.pallas_call(
        _mlp_body,
        grid=(grid,),
        in_specs=[
            col_spec(GMF_DIM), col_spec(GMF_DIM),
            col_spec(MLP_DIM), col_spec(MLP_DIM),
            full((MLP_DIM, MLP_DIM)), full((MLP_DIM, MLP_DIM)),
            full((MLP_DIM, 1)),
            full((32, MLP_DIM)), full((32, 1)),
            full((GMF_DIM, 32)), full((GMF_DIM, 1)),
            full((GMF_DIM, 1)), full((GMF_DIM, 1)), full((1,)),
        ],
        out_specs=pl.BlockSpec((blk,), lambda i: (i,)),
        out_shape=jax.ShapeDtypeStruct((B,), jnp.float32),
    )(
        ug, ig, um, im,
        W1[:, :MLP_DIM], W1[:, MLP_DIM:], b1.reshape(-1, 1),
        W2, b2.reshape(-1, 1),
        W3, b3.reshape(-1, 1),
        Wp[:, :GMF_DIM].reshape(-1, 1), Wp[:, GMF_DIM:].reshape(-1, 1), bp,
    )
    return out
